# baseline (speedup 1.0000x reference)
import functools

import jax
import jax.numpy as jnp
from jax import lax
from jax.experimental import pallas as pl
from jax.experimental.pallas import tpu as pltpu
from jax.experimental.pallas import tpu_sc as plsc

N = 10000
E = 320000
D = 128
H1, O1 = 8, 8
H2, O2 = 1, 7

K = 128
CPW = 80
E_PAD = 32 * CPW * K
NP = 10240
RPT = 624

C1 = 80
C2 = 16

_BLK = 512
_GRID = (N + _BLK - 1) // _BLK


def _bcast_idx(c):
    return (lax.broadcasted_iota(jnp.int32, (16,), 0) >> 3) + 2 * c


_GDN = lax.GatherDimensionNumbers(
    offset_dims=(), collapsed_slice_dims=(0,), start_index_map=(0,))


def _lane_gather(v, idx):
    return lax.gather(v, idx[:, None], _GDN, (1,),
                      mode=lax.GatherScatterMode.PROMISE_IN_BOUNDS)


def _acc_zero(pay, acc, sid, cols):
    def zb(i, _):
        for j in range(cols // 16):
            pay[i, pl.ds(16 * j, 16)] = jnp.zeros((16,), jnp.float32)
        return 0
    lax.fori_loop(0, K, zb, 0)
    base = sid * RPT
    for t in range(4):
        pltpu.sync_copy(pay, acc.at[pl.ds(base + K * t, K)])
    pltpu.sync_copy(pay.at[pl.ds(0, RPT - 4 * K)],
                    acc.at[pl.ds(base + 4 * K, RPT - 4 * K)])

    @pl.when(sid == 15)
    def _():
        pltpu.sync_copy(pay, acc.at[pl.ds(base + RPT, K)])
        pltpu.sync_copy(pay, acc.at[pl.ds(base + RPT + K, K)])


def _acc_out(acc, out_hbm, cid, sid):
    base = sid * RPT
    pltpu.sync_copy(acc.at[pl.ds(base, RPT)],
                    out_hbm.at[cid, pl.ds(base, RPT)])

    @pl.when(sid == 15)
    def _():
        pltpu.sync_copy(acc.at[pl.ds(base + RPT, 16)],
                        out_hbm.at[cid, pl.ds(base + RPT, 16)])


def _edge_pipeline(ps_hbm, pd_hbm, src_hbm, dst_hbm,
                   sbuf, dbuf, sg, dg, pay, acc, gs, gd, ss,
                   cid, sid, compute_chunk):
    wid = sid * 2 + cid
    lo = wid * CPW
    pltpu.sync_copy(src_hbm.at[pl.ds(lo, CPW)], sbuf)
    pltpu.sync_copy(dst_hbm.at[pl.ds(lo, CPW)], dbuf)

    def start_gather(j, b):
        pltpu.async_copy(ps_hbm.at[sbuf.at[j]], sg[b], gs[b])
        pltpu.async_copy(pd_hbm.at[dbuf.at[j]], dg[b], gd[b])

    def wait_gather(b):
        pltpu.make_async_copy(ps_hbm.at[sbuf.at[0]], sg[b], gs[b]).wait()
        pltpu.make_async_copy(pd_hbm.at[dbuf.at[0]], dg[b], gd[b]).wait()

    def wait_scatter(b):
        pltpu.make_async_copy(pay[b], acc.at[dbuf.at[0]], ss[b]).wait()

    start_gather(0, 0)

    def pair(p, _):
        for b in range(2):
            j = p * 2 + b
            wait_gather(b)

            @pl.when(j + 1 < CPW)
            def _(j=j, b=b):
                start_gather(j + 1, 1 - b)

            @pl.when(j >= 2)
            def _(b=b):
                wait_scatter(b)
            compute_chunk(sg[b], dg[b], pay[b])
            pltpu.async_copy(pay[b], acc.at[dbuf.at[j]], ss[b], add=True)
        return 0
    lax.fori_loop(0, CPW // 2, pair, 0)

    for b in range(2):
        wait_scatter(b)


def _sc_edge1(ps_hbm, pd_hbm, src_hbm, dst_hbm, out_hbm,
              sbuf, dbuf, sg0, sg1, dg0, dg1, pay0, pay1,
              acc, gs0, gs1, gd0, gd1, ss0, ss1):
    cid = lax.axis_index("c")
    sid = lax.axis_index("s")
    _acc_zero(pay0, acc, sid, C1)
    plsc.subcore_barrier()

    def compute(sg, dg, pay):
        @plsc.parallel_loop(0, K, unroll=4)
        def edge(k):
            t = sg[k, pl.ds(64, 16)] + dg[k]
            w = jnp.exp(jnp.maximum(t, 0.2 * t))
            pay[k, pl.ds(64, 16)] = w
            for c in range(4):
                we = _lane_gather(w, _bcast_idx(c))
                pay[k, pl.ds(16 * c, 16)] = sg[k, pl.ds(16 * c, 16)] * we

    _edge_pipeline(ps_hbm, pd_hbm, src_hbm, dst_hbm,
                   sbuf, dbuf, [sg0, sg1], [dg0, dg1],
                   [pay0, pay1], acc, [gs0, gs1], [gd0, gd1], [ss0, ss1],
                   cid, sid, compute)

    plsc.subcore_barrier()
    _acc_out(acc, out_hbm, cid, sid)


def _sc_edge2(ps_hbm, pd_hbm, src_hbm, dst_hbm, out_hbm,
              sbuf, dbuf, sg0, sg1, dg0, dg1, pay0, pay1,
              acc, gs0, gs1, gd0, gd1, ss0, ss1):
    cid = lax.axis_index("c")
    sid = lax.axis_index("s")
    _acc_zero(pay0, acc, sid, C2)
    plsc.subcore_barrier()

    idx8 = jnp.full((16,), 8, jnp.int32)

    def compute(sg, dg, pay):
        @plsc.parallel_loop(0, K, unroll=8)
        def edge(k):
            s = sg[k]
            t = s + dg[k]
            u = jnp.exp(jnp.maximum(t, 0.2 * t))
            w2 = _lane_gather(u, idx8)
            pay[k] = s * w2

    _edge_pipeline(ps_hbm, pd_hbm, src_hbm, dst_hbm,
                   sbuf, dbuf, [sg0, sg1], [dg0, dg1],
                   [pay0, pay1], acc, [gs0, gs1], [gd0, gd1], [ss0, ss1],
                   cid, sid, compute)

    plsc.subcore_barrier()
    _acc_out(acc, out_hbm, cid, sid)


_SC_MESH = plsc.VectorSubcoreMesh(core_axis_name="c", subcore_axis_name="s")


def _make_edge_pass(body, c):
    return pl.kernel(
        body,
        out_type=jax.ShapeDtypeStruct((2, N, c), jnp.float32),
        mesh=_SC_MESH,
        compiler_params=pltpu.CompilerParams(use_tc_tiling_on_sc=False),
        scratch_types=(
            [pltpu.VMEM((CPW, K), jnp.int32)] * 2
            + [pltpu.VMEM((K, c), jnp.float32)] * 2
            + [pltpu.VMEM((K, 16), jnp.float32)] * 2
            + [pltpu.VMEM((K, c), jnp.float32)] * 2
            + [pltpu.VMEM_SHARED((NP, c), jnp.float32)]
            + [pltpu.SemaphoreType.DMA] * 6
        ),
    )


def _row_mask(i):
    rows = i * _BLK + lax.broadcasted_iota(jnp.int32, (_BLK, 1), 0)
    return (rows < N).astype(jnp.float32)


def _dense1_body(x_ref, wp_ref, wd_ref, ps_ref, pd_ref):
    m = _row_mask(pl.program_id(0))
    xb = x_ref[...] * m
    ps_ref[...] = jnp.dot(xb, wp_ref[...], preferred_element_type=jnp.float32)
    pd_ref[...] = jnp.dot(xb, wd_ref[...], preferred_element_type=jnp.float32)


def _dense2_body(a0_ref, a1_ref, r_ref, b1_ref, w2p_ref, w2d_ref, e7_ref,
                 ps2_ref, pd2_ref):
    m = _row_mask(pl.program_id(0))
    a = (a0_ref[...] + a1_ref[...]) * m
    num = a[:, :64]
    den = a[:, 64:72]
    rep = jnp.dot(1.0 / (den + 1e-16), r_ref[...],
                  preferred_element_type=jnp.float32)
    o1 = (num * rep + b1_ref[...]) * m
    o1 = jnp.where(o1 > 0, o1, jnp.exp(jnp.minimum(o1, 0.0)) - 1.0)
    ps2_ref[...] = jnp.dot(o1, w2p_ref[...],
                           preferred_element_type=jnp.float32) + e7_ref[...]
    pd2_ref[...] = jnp.dot(o1, w2d_ref[...],
                           preferred_element_type=jnp.float32)


def _dense3_body(a0_ref, a1_ref, b2_ref, out_ref):
    a = a0_ref[...] + a1_ref[...]
    o = a[:, :7] / (a[:, 7:8] + 1e-16) + b2_ref[...]
    m = jnp.max(o, axis=1, keepdims=True)
    s = o - m
    out_ref[...] = s - jnp.log(jnp.sum(jnp.exp(s), axis=1, keepdims=True))


def _full(shape):
    return pl.BlockSpec(shape, lambda i: (0,) * len(shape))


def kernel(x, edge_index, W1, a_src1, a_dst1, b1, W2, a_src2, a_dst2, b2):
    npad = E_PAD - E
    pad_s = jnp.full((npad,), N, jnp.int32)
    pad_d = N + (jnp.arange(npad, dtype=jnp.int32) % 224)
    src = jnp.concatenate([edge_index[0].astype(jnp.int32), pad_s]
                          ).reshape(32 * CPW, K)
    dst = jnp.concatenate([edge_index[1].astype(jnp.int32), pad_d]
                          ).reshape(32 * CPW, K)

    Asrc = (a_src1[:, :, None] * jnp.eye(H1, dtype=jnp.float32)[:, None, :]
            ).reshape(H1 * O1, H1)
    Adst = (a_dst1[:, :, None] * jnp.eye(H1, dtype=jnp.float32)[:, None, :]
            ).reshape(H1 * O1, H1)
    P = jnp.concatenate([jnp.eye(64, dtype=jnp.float32), Asrc, Asrc], axis=1)
    W1P = W1 @ P
    W1D = W1 @ jnp.concatenate([Adst, Adst], axis=1)
    R = jnp.kron(jnp.eye(8, dtype=jnp.float32),
                 jnp.ones((1, 8), jnp.float32))
    s2 = W2 @ a_src2.reshape(O2, 1)
    d2 = W2 @ a_dst2.reshape(O2, 1)
    W2P = jnp.concatenate(
        [W2, jnp.zeros((64, 1), jnp.float32), s2 @ jnp.ones((1, 8), jnp.float32)],
        axis=1)
    W2D = d2 @ jnp.ones((1, 16), jnp.float32)
    e7 = jnp.zeros((1, 16), jnp.float32).at[0, 7].set(1.0)
    b1r = b1.reshape(1, 64)
    b2r = b2.reshape(1, O2)

    ps1, pd1 = pl.pallas_call(
        _dense1_body,
        grid=(_GRID,),
        in_specs=[pl.BlockSpec((_BLK, D), lambda i: (i, 0)),
                  _full((D, C1)), _full((D, C2))],
        out_specs=[pl.BlockSpec((_BLK, C1), lambda i: (i, 0)),
                   pl.BlockSpec((_BLK, C2), lambda i: (i, 0))],
        out_shape=[jax.ShapeDtypeStruct((NP, C1), jnp.float32),
                   jax.ShapeDtypeStruct((NP, C2), jnp.float32)],
    )(x, W1P, W1D)

    acc1 = _make_edge_pass(_sc_edge1, C1)(ps1, pd1, src, dst)

    ps2, pd2 = pl.pallas_call(
        _dense2_body,
        grid=(_GRID,),
        in_specs=[pl.BlockSpec((_BLK, C1), lambda i: (i, 0)),
                  pl.BlockSpec((_BLK, C1), lambda i: (i, 0)),
                  _full((H1, 64)), _full((1, 64)),
                  _full((64, C2)), _full((64, C2)), _full((1, C2))],
        out_specs=[pl.BlockSpec((_BLK, C2), lambda i: (i, 0)),
                   pl.BlockSpec((_BLK, C2), lambda i: (i, 0))],
        out_shape=[jax.ShapeDtypeStruct((NP, C2), jnp.float32),
                   jax.ShapeDtypeStruct((NP, C2), jnp.float32)],
    )(acc1[0], acc1[1], R, b1r, W2P, W2D, e7)

    acc2 = _make_edge_pass(_sc_edge2, C2)(ps2, pd2, src, dst)

    out = pl.pallas_call(
        _dense3_body,
        grid=(_GRID,),
        in_specs=[pl.BlockSpec((_BLK, C2), lambda i: (i, 0)),
                  pl.BlockSpec((_BLK, C2), lambda i: (i, 0)),
                  _full((1, O2))],
        out_specs=pl.BlockSpec((_BLK, O2), lambda i: (i, 0)),
        out_shape=jax.ShapeDtypeStruct((N, O2), jnp.float32),
    )(acc2[0], acc2[1], b2r)
    return out


# speedup vs baseline: 2.1242x; 2.1242x over previous
import functools

import jax
import jax.numpy as jnp
from jax import lax
from jax.experimental import pallas as pl
from jax.experimental.pallas import tpu as pltpu
from jax.experimental.pallas import tpu_sc as plsc

N = 10000
E = 320000
D = 128
H1, O1 = 8, 8
H2, O2 = 1, 7

K = 128
CPW = 80
E_PAD = 32 * CPW * K
NP = 10240
RPT = 624

C1 = 80
C2 = 16

_BLK = 512
_GRID = (N + _BLK - 1) // _BLK


def _bcast_idx(c):
    return (lax.broadcasted_iota(jnp.int32, (16,), 0) >> 3) + 2 * c


_GDN = lax.GatherDimensionNumbers(
    offset_dims=(), collapsed_slice_dims=(0,), start_index_map=(0,))


def _lane_gather(v, idx):
    return lax.gather(v, idx[:, None], _GDN, (1,),
                      mode=lax.GatherScatterMode.PROMISE_IN_BOUNDS)


def _acc_zero(pay, acc, sid, cols):
    def zb(i, _):
        for j in range(cols // 16):
            pay[i, pl.ds(16 * j, 16)] = jnp.zeros((16,), jnp.float32)
        return 0
    lax.fori_loop(0, K, zb, 0)
    base = sid * RPT
    for t in range(4):
        pltpu.sync_copy(pay, acc.at[pl.ds(base + K * t, K)])
    pltpu.sync_copy(pay.at[pl.ds(0, RPT - 4 * K)],
                    acc.at[pl.ds(base + 4 * K, RPT - 4 * K)])

    @pl.when(sid == 15)
    def _():
        pltpu.sync_copy(pay, acc.at[pl.ds(base + RPT, K)])
        pltpu.sync_copy(pay, acc.at[pl.ds(base + RPT + K, K)])


def _acc_out(acc, out_hbm, cid, sid):
    base = sid * RPT
    pltpu.sync_copy(acc.at[pl.ds(base, RPT)],
                    out_hbm.at[cid, pl.ds(base, RPT)])

    @pl.when(sid == 15)
    def _():
        pltpu.sync_copy(acc.at[pl.ds(base + RPT, 16)],
                        out_hbm.at[cid, pl.ds(base + RPT, 16)])


def _edge_pipeline(ps_hbm, pd_hbm, src_hbm, dst_hbm,
                   sbuf, dbuf, sg, dg, pay, acc, gs, gd, ss,
                   cid, sid, compute_chunk):
    wid = sid * 2 + cid
    lo = wid * CPW
    pltpu.sync_copy(src_hbm.at[pl.ds(lo, CPW)], sbuf)
    pltpu.sync_copy(dst_hbm.at[pl.ds(lo, CPW)], dbuf)

    def start_gather(j, b):
        pltpu.async_copy(ps_hbm.at[sbuf.at[j]], sg[b], gs[b])
        pltpu.async_copy(pd_hbm.at[dbuf.at[j]], dg[b], gd[b])

    def wait_gather(b):
        pltpu.make_async_copy(ps_hbm.at[sbuf.at[0]], sg[b], gs[b]).wait()
        pltpu.make_async_copy(pd_hbm.at[dbuf.at[0]], dg[b], gd[b]).wait()

    def wait_scatter(b):
        pltpu.make_async_copy(pay[b], acc.at[dbuf.at[0]], ss[b]).wait()

    start_gather(0, 0)

    def pair(p, _):
        for b in range(2):
            j = p * 2 + b
            wait_gather(b)

            @pl.when(j + 1 < CPW)
            def _(j=j, b=b):
                start_gather(j + 1, 1 - b)

            @pl.when(j >= 2)
            def _(b=b):
                wait_scatter(b)
            compute_chunk(sg[b], dg[b], pay[b])
            pltpu.async_copy(pay[b], acc.at[dbuf.at[j]], ss[b], add=True)
        return 0
    lax.fori_loop(0, CPW // 2, pair, 0)

    for b in range(2):
        wait_scatter(b)


def _sc_edge1(ps_hbm, pd_hbm, src_hbm, dst_hbm, out_hbm,
              sbuf, dbuf, sg0, sg1, dg0, dg1, pay0, pay1,
              acc, gs0, gs1, gd0, gd1, ss0, ss1):
    cid = lax.axis_index("c")
    sid = lax.axis_index("s")
    _acc_zero(pay0, acc, sid, C1)
    plsc.subcore_barrier()

    def compute(sg, dg, pay):
        @plsc.parallel_loop(0, K, unroll=4)
        def edge(k):
            t = sg[k, pl.ds(64, 16)] + dg[k]
            w = jnp.exp(jnp.maximum(t, 0.2 * t))
            pay[k, pl.ds(64, 16)] = w
            for c in range(4):
                we = _lane_gather(w, _bcast_idx(c))
                pay[k, pl.ds(16 * c, 16)] = sg[k, pl.ds(16 * c, 16)] * we

    _edge_pipeline(ps_hbm, pd_hbm, src_hbm, dst_hbm,
                   sbuf, dbuf, [sg0, sg1], [dg0, dg1],
                   [pay0, pay1], acc, [gs0, gs1], [gd0, gd1], [ss0, ss1],
                   cid, sid, compute)

    plsc.subcore_barrier()
    _acc_out(acc, out_hbm, cid, sid)


def _sc_edge2(ps_hbm, pd_hbm, src_hbm, dst_hbm, out_hbm,
              sbuf, dbuf, sg0, sg1, dg0, dg1, pay0, pay1,
              acc, gs0, gs1, gd0, gd1, ss0, ss1):
    cid = lax.axis_index("c")
    sid = lax.axis_index("s")
    _acc_zero(pay0, acc, sid, C2)
    plsc.subcore_barrier()

    idx8 = jnp.full((16,), 8, jnp.int32)

    def compute(sg, dg, pay):
        @plsc.parallel_loop(0, K, unroll=8)
        def edge(k):
            s = sg[k]
            t = s + dg[k]
            u = jnp.exp(jnp.maximum(t, 0.2 * t))
            w2 = _lane_gather(u, idx8)
            pay[k] = s * w2

    _edge_pipeline(ps_hbm, pd_hbm, src_hbm, dst_hbm,
                   sbuf, dbuf, [sg0, sg1], [dg0, dg1],
                   [pay0, pay1], acc, [gs0, gs1], [gd0, gd1], [ss0, ss1],
                   cid, sid, compute)

    plsc.subcore_barrier()
    _acc_out(acc, out_hbm, cid, sid)


_SC_MESH = plsc.VectorSubcoreMesh(core_axis_name="c", subcore_axis_name="s")


def _make_edge_pass(body, c):
    return pl.kernel(
        body,
        out_type=jax.ShapeDtypeStruct((2, N, c), jnp.float32),
        mesh=_SC_MESH,
        compiler_params=pltpu.CompilerParams(use_tc_tiling_on_sc=False),
        scratch_types=(
            [pltpu.VMEM((CPW, K), jnp.int32)] * 2
            + [pltpu.VMEM((K, c), jnp.float32)] * 2
            + [pltpu.VMEM((K, 16), jnp.float32)] * 2
            + [pltpu.VMEM((K, c), jnp.float32)] * 2
            + [pltpu.VMEM_SHARED((NP, c), jnp.float32)]
            + [pltpu.SemaphoreType.DMA] * 6
        ),
    )


def _row_mask(i):
    rows = i * _BLK + lax.broadcasted_iota(jnp.int32, (_BLK, 1), 0)
    return (rows < N).astype(jnp.float32)


def _dense1_body(x_ref, wp_ref, wd_ref, ps_ref, pd_ref):
    m = _row_mask(pl.program_id(0))
    xb = x_ref[...] * m
    ps_ref[...] = jnp.dot(xb, wp_ref[...], preferred_element_type=jnp.float32)
    pd_ref[...] = jnp.dot(xb, wd_ref[...], preferred_element_type=jnp.float32)


def _dense2_body(a0_ref, a1_ref, r_ref, b1_ref, w2p_ref, w2d_ref, e7_ref,
                 ps2_ref, pd2_ref):
    m = _row_mask(pl.program_id(0))
    a = (a0_ref[...] + a1_ref[...]) * m
    num = a[:, :64]
    den = a[:, 64:72]
    rep = jnp.dot(1.0 / (den + 1e-16), r_ref[...],
                  preferred_element_type=jnp.float32)
    o1 = (num * rep + b1_ref[...]) * m
    o1 = jnp.where(o1 > 0, o1, jnp.exp(jnp.minimum(o1, 0.0)) - 1.0)
    ps2_ref[...] = jnp.dot(o1, w2p_ref[...],
                           preferred_element_type=jnp.float32) + e7_ref[...]
    pd2_ref[...] = jnp.dot(o1, w2d_ref[...],
                           preferred_element_type=jnp.float32)


def _dense3_body(a0_ref, a1_ref, b2_ref, out_ref):
    a = a0_ref[...] + a1_ref[...]
    o = a[:, :7] / (a[:, 7:8] + 1e-16) + b2_ref[...]
    m = jnp.max(o, axis=1, keepdims=True)
    s = o - m
    out_ref[...] = s - jnp.log(jnp.sum(jnp.exp(s), axis=1, keepdims=True))


def _full(shape):
    return pl.BlockSpec(shape, lambda i: (0,) * len(shape))


def kernel(x, edge_index, W1, a_src1, a_dst1, b1, W2, a_src2, a_dst2, b2):
    npad = E_PAD - E
    pad_s = N + (jnp.arange(npad, dtype=jnp.int32) % 224)
    pad_d = N + (jnp.arange(npad, dtype=jnp.int32) % 224)
    src = jnp.concatenate([edge_index[0].astype(jnp.int32), pad_s]
                          ).reshape(32 * CPW, K)
    dst = jnp.concatenate([edge_index[1].astype(jnp.int32), pad_d]
                          ).reshape(32 * CPW, K)

    Asrc = (a_src1[:, :, None] * jnp.eye(H1, dtype=jnp.float32)[:, None, :]
            ).reshape(H1 * O1, H1)
    Adst = (a_dst1[:, :, None] * jnp.eye(H1, dtype=jnp.float32)[:, None, :]
            ).reshape(H1 * O1, H1)
    P = jnp.concatenate([jnp.eye(64, dtype=jnp.float32), Asrc, Asrc], axis=1)
    W1P = W1 @ P
    W1D = W1 @ jnp.concatenate([Adst, Adst], axis=1)
    R = jnp.kron(jnp.eye(8, dtype=jnp.float32),
                 jnp.ones((1, 8), jnp.float32))
    s2 = W2 @ a_src2.reshape(O2, 1)
    d2 = W2 @ a_dst2.reshape(O2, 1)
    W2P = jnp.concatenate(
        [W2, jnp.zeros((64, 1), jnp.float32), s2 @ jnp.ones((1, 8), jnp.float32)],
        axis=1)
    W2D = d2 @ jnp.ones((1, 16), jnp.float32)
    e7 = jnp.zeros((1, 16), jnp.float32).at[0, 7].set(1.0)
    b1r = b1.reshape(1, 64)
    b2r = b2.reshape(1, O2)

    ps1, pd1 = pl.pallas_call(
        _dense1_body,
        grid=(_GRID,),
        in_specs=[pl.BlockSpec((_BLK, D), lambda i: (i, 0)),
                  _full((D, C1)), _full((D, C2))],
        out_specs=[pl.BlockSpec((_BLK, C1), lambda i: (i, 0)),
                   pl.BlockSpec((_BLK, C2), lambda i: (i, 0))],
        out_shape=[jax.ShapeDtypeStruct((NP, C1), jnp.float32),
                   jax.ShapeDtypeStruct((NP, C2), jnp.float32)],
    )(x, W1P, W1D)

    acc1 = _make_edge_pass(_sc_edge1, C1)(ps1, pd1, src, dst)

    ps2, pd2 = pl.pallas_call(
        _dense2_body,
        grid=(_GRID,),
        in_specs=[pl.BlockSpec((_BLK, C1), lambda i: (i, 0)),
                  pl.BlockSpec((_BLK, C1), lambda i: (i, 0)),
                  _full((H1, 64)), _full((1, 64)),
                  _full((64, C2)), _full((64, C2)), _full((1, C2))],
        out_specs=[pl.BlockSpec((_BLK, C2), lambda i: (i, 0)),
                   pl.BlockSpec((_BLK, C2), lambda i: (i, 0))],
        out_shape=[jax.ShapeDtypeStruct((NP, C2), jnp.float32),
                   jax.ShapeDtypeStruct((NP, C2), jnp.float32)],
    )(acc1[0], acc1[1], R, b1r, W2P, W2D, e7)

    acc2 = _make_edge_pass(_sc_edge2, C2)(ps2, pd2, src, dst)

    out = pl.pallas_call(
        _dense3_body,
        grid=(_GRID,),
        in_specs=[pl.BlockSpec((_BLK, C2), lambda i: (i, 0)),
                  pl.BlockSpec((_BLK, C2), lambda i: (i, 0)),
                  _full((1, O2))],
        out_specs=pl.BlockSpec((_BLK, O2), lambda i: (i, 0)),
        out_shape=jax.ShapeDtypeStruct((N, O2), jnp.float32),
    )(acc2[0], acc2[1], b2r)
    return out
